# baseline (speedup 1.0000x reference)
import jax
import jax.numpy as jnp
from jax.experimental import pallas as pl
from jax.experimental.pallas import tpu as pltpu

EMBED_DIM = 128
SPATIAL = 32 * 32
BB = 8
NBUF = 8


def _gather_body(emb_ref, t_ref, g_ref):
    vpad = emb_ref.shape[0]
    b = t_ref.shape[0]
    cols = jax.lax.broadcasted_iota(jnp.int32, (b, vpad), 1)
    onehot = (cols == t_ref[:, 0][:, None]).astype(jnp.float32)
    g_ref[...] = jax.lax.dot_general(
        onehot, emb_ref[...], (((1,), (0,)), ((), ())),
        preferred_element_type=jnp.float32)


def _broadcast_body(g_ref, o_hbm, scratch, sems):
    i = pl.program_id(0)
    nsteps = pl.num_programs(0)
    for k in range(NBUF):
        chunk = i * NBUF + k
        buf = scratch.at[pl.ds(k * BB, BB)]
        cp = pltpu.make_async_copy(
            buf, o_hbm.at[pl.ds(chunk * BB, BB)], sems.at[k])

        @pl.when(i > 0)
        def _():
            cp.wait()

        gt = jnp.swapaxes(g_ref[pl.ds(chunk * BB, BB), :], 0, 1)
        for j in range(BB):
            scratch[k * BB + j] = jnp.broadcast_to(
                gt[:, j:j + 1], (EMBED_DIM, SPATIAL))
        cp.start()

    @pl.when(i == nsteps - 1)
    def _():
        for k in range(NBUF):
            pltpu.make_async_copy(
                scratch.at[pl.ds(k * BB, BB)],
                o_hbm.at[pl.ds(k * BB, BB)], sems.at[k]).wait()


def kernel(x, t, embedding):
    B = t.shape[0]
    V = embedding.shape[0]
    vpad = (V + 7) // 8 * 8
    emb_pad = jnp.pad(embedding, ((0, vpad - V), (0, 0)))

    g = pl.pallas_call(
        _gather_body,
        out_shape=jax.ShapeDtypeStruct((B, EMBED_DIM), jnp.float32),
    )(emb_pad, t.reshape(B, 1))

    out = pl.pallas_call(
        _broadcast_body,
        grid=(B // (BB * NBUF),),
        in_specs=[pl.BlockSpec((B, EMBED_DIM), lambda i: (0, 0))],
        out_specs=pl.BlockSpec(memory_space=pl.ANY),
        out_shape=jax.ShapeDtypeStruct((B, EMBED_DIM, SPATIAL), jnp.float32),
        scratch_shapes=[
            pltpu.VMEM((NBUF * BB, EMBED_DIM, SPATIAL), jnp.float32),
            pltpu.SemaphoreType.DMA((NBUF,)),
        ],
        compiler_params=pltpu.CompilerParams(
            dimension_semantics=("arbitrary",)),
    )(g)
    return out.reshape(B, EMBED_DIM, x.shape[-2], x.shape[-1])
